# baseline (speedup 1.0000x reference)
import functools

import jax
import jax.numpy as jnp
from jax.experimental import pallas as pl

N = 10000
NFEAT = 128
NHID = 64
NCLASS = 64
BM = 400


def _l2n(h):
    n = jnp.sqrt(jnp.sum(h * h, axis=1, keepdims=True))
    return h / jnp.maximum(n, 1e-12)


def _pass1_body(adj_ref, xb_ref, xs_ref, ws_ref, bs_ref, wn_ref, bn_ref,
                h1f_ref, h1b_ref):
    adj_bf = adj_ref[...].astype(jnp.bfloat16)
    agg = jnp.dot(adj_bf, xb_ref[...], preferred_element_type=jnp.float32)
    hs = jnp.dot(xs_ref[...], ws_ref[...],
                 preferred_element_type=jnp.float32) + bs_ref[...]
    hn = jnp.dot(agg, wn_ref[...],
                 preferred_element_type=jnp.float32) + bn_ref[...]
    h = jax.nn.relu(jnp.concatenate([hs, hn], axis=1))
    h = _l2n(h)
    h1f_ref[...] = h
    h1b_ref[...] = h.astype(jnp.bfloat16)


def _pass2_body(adj_ref, hb_ref, hf_ref, ws_ref, bs_ref, wn_ref, bn_ref,
                wfc_ref, bfc_ref, out_ref):
    adj_bf = adj_ref[...].astype(jnp.bfloat16)
    agg = jnp.dot(adj_bf, hb_ref[...], preferred_element_type=jnp.float32)
    hs = jnp.dot(hf_ref[...], ws_ref[...],
                 preferred_element_type=jnp.float32) + bs_ref[...]
    hn = jnp.dot(agg, wn_ref[...],
                 preferred_element_type=jnp.float32) + bn_ref[...]
    h = jax.nn.relu(jnp.concatenate([hs, hn], axis=1))
    h = _l2n(h)
    out_ref[...] = jnp.dot(h, wfc_ref[...],
                           preferred_element_type=jnp.float32) + bfc_ref[...]


def _row_blk(w):
    return pl.BlockSpec((BM, w), lambda i: (i, 0))


def _full(shape):
    return pl.BlockSpec(shape, lambda i: (0,) * len(shape))


@functools.partial(jax.jit, static_argnames=("interpret",))
def _run(x, adj, W1s, b1s, W1n, b1n, W2s, b2s, W2n, b2n, Wfc, bfc,
         interpret=False):
    grid = (N // BM,)
    xb = x.astype(jnp.bfloat16)
    b1s2 = b1s.reshape(1, NHID)
    b1n2 = b1n.reshape(1, NHID)
    b2s2 = b2s.reshape(1, NHID)
    b2n2 = b2n.reshape(1, NHID)
    bfc2 = bfc.reshape(1, NCLASS)

    h1f, h1b = pl.pallas_call(
        _pass1_body,
        grid=grid,
        in_specs=[
            _row_blk(N),
            _full((N, NFEAT)),
            _row_blk(NFEAT),
            _full((NFEAT, NHID)),
            _full((1, NHID)),
            _full((NFEAT, NHID)),
            _full((1, NHID)),
        ],
        out_specs=[_row_blk(2 * NHID), _row_blk(2 * NHID)],
        out_shape=[
            jax.ShapeDtypeStruct((N, 2 * NHID), jnp.float32),
            jax.ShapeDtypeStruct((N, 2 * NHID), jnp.bfloat16),
        ],
        interpret=interpret,
    )(adj, xb, x, W1s, b1s2, W1n, b1n2)

    out = pl.pallas_call(
        _pass2_body,
        grid=grid,
        in_specs=[
            _row_blk(N),
            _full((N, 2 * NHID)),
            _row_blk(2 * NHID),
            _full((2 * NHID, NHID)),
            _full((1, NHID)),
            _full((2 * NHID, NHID)),
            _full((1, NHID)),
            _full((2 * NHID, NCLASS)),
            _full((1, NCLASS)),
        ],
        out_specs=_row_blk(NCLASS),
        out_shape=jax.ShapeDtypeStruct((N, NCLASS), jnp.float32),
        interpret=interpret,
    )(adj, h1b, h1f, W2s, b2s2, W2n, b2n2, Wfc, bfc2)
    return out


def kernel(x, adj, W1s, b1s, W1n, b1n, W2s, b2s, W2n, b2n, Wfc, bfc):
    return _run(x, adj, W1s, b1s, W1n, b1n, W2s, b2s, W2n, b2n, Wfc, bfc)
